# baseline (speedup 1.0000x reference)
import functools

import jax
import jax.numpy as jnp
from jax import lax
from jax.experimental import pallas as pl
from jax.experimental.pallas import tpu as pltpu
from jax.experimental.pallas import tpu_sc as plsc

NC = 2
NS = 16
NW = NC * NS
CK = 128
NR = 4


def _pad_rows(n):
    q = 128 * NS
    return (n + q - 1) // q * q


def _sc_mesh():
    return plsc.VectorSubcoreMesh(core_axis_name="c", subcore_axis_name="s")


def _edges_pad(src, dst, n, npad):
    e = src.shape[0]
    nch = -(-e // (NW * CK))
    pad = NW * nch * CK - e
    if pad:
        ar = jnp.arange(pad, dtype=jnp.int32)
        src = jnp.concatenate([src, ar % n])
        dst = jnp.concatenate([dst, n + ar % (npad - n)])
    return src, dst, nch


def _deg_call(dst1, zeros1, ones1, nch, npad):
    rpt = npad // NS

    @functools.partial(
        pl.kernel,
        out_type=jax.ShapeDtypeStruct((NC * npad,), jnp.float32),
        mesh=_sc_mesh(),
        scratch_types=[
            pltpu.VMEM((NR, CK), jnp.int32),
            pltpu.VMEM((CK,), jnp.float32),
            pltpu.VMEM_SHARED((npad,), jnp.float32),
            pltpu.SemaphoreType.DMA,
            pltpu.SemaphoreType.DMA,
            pltpu.SemaphoreType.DMA,
            pltpu.SemaphoreType.DMA,
            pltpu.SemaphoreType.DMA,
            pltpu.SemaphoreType.DMA,
            pltpu.SemaphoreType.DMA,
            pltpu.SemaphoreType.DMA,
        ],
    )
    def deg_kernel(dst_hbm, z_hbm, ones_hbm, out_hbm, dring, ones_v, acc_sh,
                   i0, i1, i2, i3, s0, s1, s2, s3):
        isem = (i0, i1, i2, i3)
        ssem = (s0, s1, s2, s3)
        cid = lax.axis_index("c")
        sid = lax.axis_index("s")
        wid = sid * NC + cid
        ebase = wid * nch * CK

        def idx_start(g, slot):
            pltpu.async_copy(dst_hbm.at[pl.ds(ebase + g * CK, CK)],
                             dring.at[slot], isem[slot])

        def idx_wait(slot):
            pltpu.make_async_copy(dst_hbm.at[pl.ds(0, CK)],
                                  dring.at[slot], isem[slot]).wait()

        def scatter_start(g, slot):
            pltpu.async_copy(ones_v, acc_sh.at[dring.at[slot]],
                             ssem[slot], add=True)

        def scatter_wait(slot):
            pltpu.make_async_copy(ones_v, acc_sh.at[dring.at[0]],
                                  ssem[slot]).wait()

        def step(g):
            slot = g % NR
            idx_wait(slot)
            scatter_start(g, slot)
            if g + 3 <= nch - 1:
                if g >= 1:
                    scatter_wait((g - 1) % NR)
                idx_start(g + 3, (g + 3) % NR)

        for g in range(min(3, nch)):
            idx_start(g, g % NR)
        pltpu.sync_copy(z_hbm.at[pl.ds(sid * rpt, rpt)],
                        acc_sh.at[pl.ds(sid * rpt, rpt)])
        pltpu.sync_copy(ones_hbm, ones_v)
        plsc.subcore_barrier()
        head = min(1, nch)
        for g in range(head):
            step(g)
        blocks = max(0, nch - 3 - head) // NR

        def body(j, carry):
            gb = head + j * NR
            for u in range(NR):
                g = gb + u
                slot = (head + u) % NR
                idx_wait(slot)
                pltpu.async_copy(ones_v, acc_sh.at[dring.at[slot]],
                                 ssem[slot], add=True)
                scatter_wait((slot + 3) % NR)
                pltpu.async_copy(dst_hbm.at[pl.ds(ebase + (g + 3) * CK, CK)],
                                 dring.at[(slot + 3) % NR],
                                 isem[(slot + 3) % NR])
            return carry

        if blocks > 0:
            lax.fori_loop(0, blocks, body, 0)
        for g in range(head + blocks * NR, nch):
            step(g)
        for t in range(max(0, nch - NR), nch):
            scatter_wait(t % NR)
        plsc.subcore_barrier()
        pltpu.sync_copy(acc_sh.at[pl.ds(sid * rpt, rpt)],
                        out_hbm.at[pl.ds(cid * npad + sid * rpt, rpt)])

    return deg_kernel(dst1, zeros1, ones1)


def _dis_call(degp3):
    _, rows, cols = degp3.shape

    def body(d_ref, o_ref):
        d = d_ref[...]
        o_ref[...] = lax.rsqrt(d[0] + d[1] + 1.0)

    return pl.pallas_call(
        body,
        out_shape=jax.ShapeDtypeStruct((rows, cols), jnp.float32),
    )(degp3)


def _msg_call(src1, dst1, hs, zeros128, nch, npad):
    d = hs.shape[1]
    rpt = npad // NS

    @functools.partial(
        pl.kernel,
        out_type=jax.ShapeDtypeStruct((NC, npad, d), jnp.float32),
        mesh=_sc_mesh(),
        scratch_types=[
            pltpu.VMEM((NR, CK), jnp.int32),
            pltpu.VMEM((NR, CK), jnp.int32),
            pltpu.VMEM((2, CK, d), jnp.float32),
            pltpu.VMEM_SHARED((npad, d), jnp.float32),
            pltpu.SemaphoreType.DMA,
            pltpu.SemaphoreType.DMA,
            pltpu.SemaphoreType.DMA,
            pltpu.SemaphoreType.DMA,
            pltpu.SemaphoreType.DMA,
            pltpu.SemaphoreType.DMA,
            pltpu.SemaphoreType.DMA,
            pltpu.SemaphoreType.DMA,
        ],
    )
    def msg_kernel(src_hbm, dst_hbm, hs_hbm, z_hbm, out_hbm,
                   sring, dring, rows_v, acc_sh,
                   i0, i1, i2, i3, g0, g1, c0, c1):
        isem = (i0, i1, i2, i3)
        gsem = (g0, g1)
        ssem = (c0, c1)
        cid = lax.axis_index("c")
        sid = lax.axis_index("s")
        wid = sid * NC + cid
        ebase = wid * nch * CK

        def idx_start(g, slot):
            pltpu.async_copy(src_hbm.at[pl.ds(ebase + g * CK, CK)],
                             sring.at[slot], isem[slot])
            pltpu.async_copy(dst_hbm.at[pl.ds(ebase + g * CK, CK)],
                             dring.at[slot], isem[slot])

        def idx_wait(slot):
            pltpu.make_async_copy(src_hbm.at[pl.ds(0, CK)],
                                  sring.at[slot], isem[slot]).wait()
            pltpu.make_async_copy(dst_hbm.at[pl.ds(0, CK)],
                                  dring.at[slot], isem[slot]).wait()

        def gather_start(g, islot, bslot):
            pltpu.async_copy(hs_hbm.at[sring.at[islot]], rows_v.at[bslot],
                             gsem[bslot])

        def gather_wait(bslot):
            pltpu.make_async_copy(hs_hbm.at[sring.at[0]],
                                  rows_v.at[bslot], gsem[bslot]).wait()

        def scatter_start(g, islot, bslot):
            pltpu.async_copy(rows_v.at[bslot], acc_sh.at[dring.at[islot]],
                             ssem[bslot], add=True)

        def scatter_wait(bslot):
            pltpu.make_async_copy(rows_v.at[bslot], acc_sh.at[dring.at[0]],
                                  ssem[bslot]).wait()

        def step(g):
            if g + 1 <= nch - 1:
                idx_wait((g + 1) % NR)
            gather_wait(g % 2)
            if g >= 1:
                scatter_wait((g + 1) % 2)
            scatter_start(g, g % NR, g % 2)
            if g + 1 <= nch - 1:
                gather_start(g + 1, (g + 1) % NR, (g + 1) % 2)
            if g + 3 <= nch - 1:
                idx_start(g + 3, (g + 3) % NR)

        for g in range(min(3, nch)):
            idx_start(g, g % NR)
        if nch > 0:
            idx_wait(0)
            gather_start(0, 0, 0)
        for z in range(rpt // CK):
            pltpu.sync_copy(z_hbm, acc_sh.at[pl.ds(sid * rpt + z * CK, CK)])
        plsc.subcore_barrier()
        head = min(1, nch)
        for g in range(head):
            step(g)
        blocks = max(0, nch - 3 - head) // NR

        def body(j, carry):
            gb = head + j * NR
            for u in range(NR):
                g = gb + u
                slot = (head + u) % NR
                nslot = (head + u + 1) % NR
                pslot = (head + u + 3) % NR
                bslot = (head + u) % 2
                idx_wait(nslot)
                gather_wait(bslot)
                scatter_wait(1 - bslot)
                pltpu.async_copy(rows_v.at[bslot], acc_sh.at[dring.at[slot]],
                                 ssem[bslot], add=True)
                pltpu.async_copy(hs_hbm.at[sring.at[nslot]],
                                 rows_v.at[1 - bslot], gsem[1 - bslot])
                pltpu.async_copy(src_hbm.at[pl.ds(ebase + (g + 3) * CK, CK)],
                                 sring.at[pslot], isem[pslot])
                pltpu.async_copy(dst_hbm.at[pl.ds(ebase + (g + 3) * CK, CK)],
                                 dring.at[pslot], isem[pslot])
            return carry

        if blocks > 0:
            lax.fori_loop(0, blocks, body, 0)
        for g in range(head + blocks * NR, nch):
            step(g)
        if nch > 0:
            scatter_wait((nch - 1) % 2)
        plsc.subcore_barrier()
        pltpu.sync_copy(acc_sh.at[pl.ds(sid * rpt, rpt)],
                        out_hbm.at[cid, pl.ds(sid * rpt, rpt)])

    return msg_kernel(src1, dst1, hs, zeros128)


def _hs_call(x, W, dis):
    n, din = x.shape
    dh = W.shape[1]
    bm = 1000

    def body(x_ref, w_ref, d_ref, o_ref):
        h = jnp.dot(x_ref[...], w_ref[...], preferred_element_type=jnp.float32)
        o_ref[...] = h * d_ref[...]

    return pl.pallas_call(
        body,
        grid=(n // bm,),
        in_specs=[
            pl.BlockSpec((bm, din), lambda i: (i, 0)),
            pl.BlockSpec((din, dh), lambda i: (0, 0)),
            pl.BlockSpec((bm, 1), lambda i: (i, 0)),
        ],
        out_specs=pl.BlockSpec((bm, dh), lambda i: (i, 0)),
        out_shape=jax.ShapeDtypeStruct((n, dh), jnp.float32),
    )(x, W, dis)


def _logits_call(msgp, hs, dis, b2, wc_pad, bc2):
    n, dh = hs.shape
    dc = wc_pad.shape[1]
    bm = 1000

    def body(m_ref, hs_ref, d_ref, b_ref, wc_ref, bc_ref, o_ref):
        m = m_ref[...]
        t = (m[0] + m[1] + hs_ref[...]) * d_ref[...] + b_ref[...]
        o_ref[...] = (
            jnp.dot(t, wc_ref[...], preferred_element_type=jnp.float32)
            + bc_ref[...]
        )

    return pl.pallas_call(
        body,
        grid=(n // bm,),
        in_specs=[
            pl.BlockSpec((NC, bm, dh), lambda i: (0, i, 0)),
            pl.BlockSpec((bm, dh), lambda i: (i, 0)),
            pl.BlockSpec((bm, 1), lambda i: (i, 0)),
            pl.BlockSpec((1, dh), lambda i: (0, 0)),
            pl.BlockSpec((dh, dc), lambda i: (0, 0)),
            pl.BlockSpec((1, dc), lambda i: (0, 0)),
        ],
        out_specs=pl.BlockSpec((bm, dc), lambda i: (i, 0)),
        out_shape=jax.ShapeDtypeStruct((n, dc), jnp.float32),
    )(msgp, hs, dis, b2, wc_pad, bc2)


def kernel(x, edge_index, W, b, Wc, bc):
    n = x.shape[0]
    dh = W.shape[1]
    num_classes = Wc.shape[1]
    src = edge_index[0].astype(jnp.int32)
    dst = edge_index[1].astype(jnp.int32)

    npad = _pad_rows(n)
    src1, dst1, nch = _edges_pad(src, dst, n, npad)
    zeros1 = jnp.zeros((npad,), jnp.float32)
    ones1 = jnp.ones((CK,), jnp.float32)
    zeros128 = jnp.zeros((CK, dh), jnp.float32)

    deg_flat = _deg_call(dst1, zeros1, ones1, nch, npad)
    dis2 = _dis_call(deg_flat.reshape(NC, npad // 128, 128))
    dis = dis2.reshape(npad)[:n].reshape(n, 1)

    hs = _hs_call(x, W, dis)
    msgp = _msg_call(src1, dst1, hs, zeros128, nch, npad)

    wc_pad = jnp.zeros((dh, dh), jnp.float32).at[:, :num_classes].set(Wc)
    bc_pad = jnp.zeros((1, dh), jnp.float32).at[0, :num_classes].set(bc)
    logits_pad = _logits_call(msgp, hs, dis, b.reshape(1, dh), wc_pad, bc_pad)
    return logits_pad[:, :num_classes]


# speedup vs baseline: 1.0045x; 1.0045x over previous
import functools

import jax
import jax.numpy as jnp
from jax import lax
from jax.experimental import pallas as pl
from jax.experimental.pallas import tpu as pltpu
from jax.experimental.pallas import tpu_sc as plsc

NC = 2
NS = 16
NW = NC * NS
CK = 128
NR = 4


def _pad_rows(n):
    q = 128 * NS
    return (n + q - 1) // q * q


def _sc_mesh():
    return plsc.VectorSubcoreMesh(core_axis_name="c", subcore_axis_name="s")


def _edges_pad(src, dst, n, npad):
    e = src.shape[0]
    nch = -(-e // (NW * CK))
    pad = NW * nch * CK - e
    if pad:
        ar = jnp.arange(pad, dtype=jnp.int32)
        src = jnp.concatenate([src, ar % n])
        dst = jnp.concatenate([dst, n + ar % (npad - n)])
    return src, dst, nch


def _deg_call(dst1, zeros1, ones1, nch, npad):
    rpt = npad // NS

    @functools.partial(
        pl.kernel,
        out_type=jax.ShapeDtypeStruct((NC * npad,), jnp.float32),
        mesh=_sc_mesh(),
        scratch_types=[
            pltpu.VMEM((NR, CK), jnp.int32),
            pltpu.VMEM((CK,), jnp.float32),
            pltpu.VMEM_SHARED((npad,), jnp.float32),
            pltpu.SemaphoreType.DMA,
            pltpu.SemaphoreType.DMA,
            pltpu.SemaphoreType.DMA,
            pltpu.SemaphoreType.DMA,
            pltpu.SemaphoreType.DMA,
            pltpu.SemaphoreType.DMA,
            pltpu.SemaphoreType.DMA,
            pltpu.SemaphoreType.DMA,
        ],
    )
    def deg_kernel(dst_hbm, z_hbm, ones_hbm, out_hbm, dring, ones_v, acc_sh,
                   i0, i1, i2, i3, s0, s1, s2, s3):
        isem = (i0, i1, i2, i3)
        ssem = (s0, s1, s2, s3)
        cid = lax.axis_index("c")
        sid = lax.axis_index("s")
        wid = sid * NC + cid
        ebase = wid * nch * CK

        def idx_start(g, slot):
            pltpu.async_copy(dst_hbm.at[pl.ds(ebase + g * CK, CK)],
                             dring.at[slot], isem[slot])

        def idx_wait(slot):
            pltpu.make_async_copy(dst_hbm.at[pl.ds(0, CK)],
                                  dring.at[slot], isem[slot]).wait()

        def scatter_start(g, slot):
            pltpu.async_copy(ones_v, acc_sh.at[dring.at[slot]],
                             ssem[slot], add=True)

        def scatter_wait(slot):
            pltpu.make_async_copy(ones_v, acc_sh.at[dring.at[0]],
                                  ssem[slot]).wait()

        def step(g):
            slot = g % NR
            idx_wait(slot)
            scatter_start(g, slot)
            if g + 3 <= nch - 1:
                if g >= 1:
                    scatter_wait((g - 1) % NR)
                idx_start(g + 3, (g + 3) % NR)

        for g in range(min(3, nch)):
            idx_start(g, g % NR)
        pltpu.sync_copy(z_hbm.at[pl.ds(sid * rpt, rpt)],
                        acc_sh.at[pl.ds(sid * rpt, rpt)])
        pltpu.sync_copy(ones_hbm, ones_v)
        plsc.subcore_barrier()
        head = min(1, nch)
        for g in range(head):
            step(g)
        blocks = max(0, nch - 3 - head) // NR

        def body(j, carry):
            gb = head + j * NR
            for u in range(NR):
                g = gb + u
                slot = (head + u) % NR
                idx_wait(slot)
                pltpu.async_copy(ones_v, acc_sh.at[dring.at[slot]],
                                 ssem[slot], add=True)
                scatter_wait((slot + 3) % NR)
                pltpu.async_copy(dst_hbm.at[pl.ds(ebase + (g + 3) * CK, CK)],
                                 dring.at[(slot + 3) % NR],
                                 isem[(slot + 3) % NR])
            return carry

        if blocks > 0:
            lax.fori_loop(0, blocks, body, 0)
        for g in range(head + blocks * NR, nch):
            step(g)
        for t in range(max(0, nch - NR), nch):
            scatter_wait(t % NR)
        plsc.subcore_barrier()
        pltpu.sync_copy(acc_sh.at[pl.ds(sid * rpt, rpt)],
                        out_hbm.at[pl.ds(cid * npad + sid * rpt, rpt)])

    return deg_kernel(dst1, zeros1, ones1)


def _dis_call(degp3):
    _, rows, cols = degp3.shape

    def body(d_ref, o_ref):
        d = d_ref[...]
        o_ref[...] = lax.rsqrt(d[0] + d[1] + 1.0)

    return pl.pallas_call(
        body,
        out_shape=jax.ShapeDtypeStruct((rows, cols), jnp.float32),
    )(degp3)


def _msg_call(src1, dst1, hs, zeros128, nch, npad):
    d = hs.shape[1]
    rpt = npad // NS

    @functools.partial(
        pl.kernel,
        out_type=jax.ShapeDtypeStruct((NC, npad, d), jnp.float32),
        mesh=_sc_mesh(),
        scratch_types=[
            pltpu.VMEM((NR, CK), jnp.int32),
            pltpu.VMEM((NR, CK), jnp.int32),
            pltpu.VMEM((2, CK, d), jnp.float32),
            pltpu.VMEM_SHARED((npad, d), jnp.float32),
            pltpu.SemaphoreType.DMA,
            pltpu.SemaphoreType.DMA,
            pltpu.SemaphoreType.DMA,
            pltpu.SemaphoreType.DMA,
            pltpu.SemaphoreType.DMA,
            pltpu.SemaphoreType.DMA,
        ],
    )
    def msg_kernel(src_hbm, dst_hbm, hs_hbm, z_hbm, out_hbm,
                   sring, dring, rows_v, acc_sh,
                   i0, i1, i2, i3, g0, g1):
        isem = (i0, i1, i2, i3)
        gsem = (g0, g1)
        cid = lax.axis_index("c")
        sid = lax.axis_index("s")
        wid = sid * NC + cid
        ebase = wid * nch * CK

        def idx_start(g, slot):
            pltpu.async_copy(src_hbm.at[pl.ds(ebase + g * CK, CK)],
                             sring.at[slot], isem[slot])
            pltpu.async_copy(dst_hbm.at[pl.ds(ebase + g * CK, CK)],
                             dring.at[slot], isem[slot])

        def idx_wait(slot):
            pltpu.make_async_copy(src_hbm.at[pl.ds(0, CK)],
                                  sring.at[slot], isem[slot]).wait()
            pltpu.make_async_copy(dst_hbm.at[pl.ds(0, CK)],
                                  dring.at[slot], isem[slot]).wait()

        def gather_start(g, islot, bslot):
            pltpu.async_copy(hs_hbm.at[sring.at[islot]], rows_v.at[bslot],
                             gsem[bslot])

        def gather_wait(bslot):
            pltpu.make_async_copy(hs_hbm.at[sring.at[0]],
                                  rows_v.at[bslot], gsem[bslot]).wait()

        def step(g):
            if g + 1 <= nch - 1:
                idx_wait((g + 1) % NR)
            gather_wait(g % 2)
            if g + 1 <= nch - 1:
                gather_start(g + 1, (g + 1) % NR, (g + 1) % 2)
            if g + 3 <= nch - 1:
                idx_start(g + 3, (g + 3) % NR)
            pltpu.sync_copy(rows_v.at[g % 2], acc_sh.at[dring.at[g % NR]],
                            add=True)

        for g in range(min(3, nch)):
            idx_start(g, g % NR)
        if nch > 0:
            idx_wait(0)
            gather_start(0, 0, 0)
        for z in range(rpt // CK):
            pltpu.sync_copy(z_hbm, acc_sh.at[pl.ds(sid * rpt + z * CK, CK)])
        plsc.subcore_barrier()
        head = min(1, nch)
        for g in range(head):
            step(g)
        blocks = max(0, nch - 3 - head) // NR

        def body(j, carry):
            gb = head + j * NR
            for u in range(NR):
                g = gb + u
                slot = (head + u) % NR
                nslot = (head + u + 1) % NR
                pslot = (head + u + 3) % NR
                bslot = (head + u) % 2
                idx_wait(nslot)
                gather_wait(bslot)
                pltpu.async_copy(hs_hbm.at[sring.at[nslot]],
                                 rows_v.at[1 - bslot], gsem[1 - bslot])
                pltpu.async_copy(src_hbm.at[pl.ds(ebase + (g + 3) * CK, CK)],
                                 sring.at[pslot], isem[pslot])
                pltpu.async_copy(dst_hbm.at[pl.ds(ebase + (g + 3) * CK, CK)],
                                 dring.at[pslot], isem[pslot])
                pltpu.sync_copy(rows_v.at[bslot], acc_sh.at[dring.at[slot]],
                                add=True)
            return carry

        if blocks > 0:
            lax.fori_loop(0, blocks, body, 0)
        for g in range(head + blocks * NR, nch):
            step(g)
        plsc.subcore_barrier()
        pltpu.sync_copy(acc_sh.at[pl.ds(sid * rpt, rpt)],
                        out_hbm.at[cid, pl.ds(sid * rpt, rpt)])

    return msg_kernel(src1, dst1, hs, zeros128)


def _hs_call(x, W, dis):
    n, din = x.shape
    dh = W.shape[1]
    bm = 1000

    def body(x_ref, w_ref, d_ref, o_ref):
        h = jnp.dot(x_ref[...], w_ref[...], preferred_element_type=jnp.float32)
        o_ref[...] = h * d_ref[...]

    return pl.pallas_call(
        body,
        grid=(n // bm,),
        in_specs=[
            pl.BlockSpec((bm, din), lambda i: (i, 0)),
            pl.BlockSpec((din, dh), lambda i: (0, 0)),
            pl.BlockSpec((bm, 1), lambda i: (i, 0)),
        ],
        out_specs=pl.BlockSpec((bm, dh), lambda i: (i, 0)),
        out_shape=jax.ShapeDtypeStruct((n, dh), jnp.float32),
    )(x, W, dis)


def _logits_call(msgp, hs, dis, b2, wc_pad, bc2):
    n, dh = hs.shape
    dc = wc_pad.shape[1]
    bm = 1000

    def body(m_ref, hs_ref, d_ref, b_ref, wc_ref, bc_ref, o_ref):
        m = m_ref[...]
        t = (m[0] + m[1] + hs_ref[...]) * d_ref[...] + b_ref[...]
        o_ref[...] = (
            jnp.dot(t, wc_ref[...], preferred_element_type=jnp.float32)
            + bc_ref[...]
        )

    return pl.pallas_call(
        body,
        grid=(n // bm,),
        in_specs=[
            pl.BlockSpec((NC, bm, dh), lambda i: (0, i, 0)),
            pl.BlockSpec((bm, dh), lambda i: (i, 0)),
            pl.BlockSpec((bm, 1), lambda i: (i, 0)),
            pl.BlockSpec((1, dh), lambda i: (0, 0)),
            pl.BlockSpec((dh, dc), lambda i: (0, 0)),
            pl.BlockSpec((1, dc), lambda i: (0, 0)),
        ],
        out_specs=pl.BlockSpec((bm, dc), lambda i: (i, 0)),
        out_shape=jax.ShapeDtypeStruct((n, dc), jnp.float32),
    )(msgp, hs, dis, b2, wc_pad, bc2)


def kernel(x, edge_index, W, b, Wc, bc):
    n = x.shape[0]
    dh = W.shape[1]
    num_classes = Wc.shape[1]
    src = edge_index[0].astype(jnp.int32)
    dst = edge_index[1].astype(jnp.int32)

    npad = _pad_rows(n)
    src1, dst1, nch = _edges_pad(src, dst, n, npad)
    zeros1 = jnp.zeros((npad,), jnp.float32)
    ones1 = jnp.ones((CK,), jnp.float32)
    zeros128 = jnp.zeros((CK, dh), jnp.float32)

    deg_flat = _deg_call(dst1, zeros1, ones1, nch, npad)
    dis2 = _dis_call(deg_flat.reshape(NC, npad // 128, 128))
    dis = dis2.reshape(npad)[:n].reshape(n, 1)

    hs = _hs_call(x, W, dis)
    msgp = _msg_call(src1, dst1, hs, zeros128, nch, npad)

    wc_pad = jnp.zeros((dh, dh), jnp.float32).at[:, :num_classes].set(Wc)
    bc_pad = jnp.zeros((1, dh), jnp.float32).at[0, :num_classes].set(bc)
    logits_pad = _logits_call(msgp, hs, dis, b.reshape(1, dh), wc_pad, bc_pad)
    return logits_pad[:, :num_classes]
